# baseline (speedup 1.0000x reference)
import jax
import jax.numpy as jnp
from jax import lax
from jax.experimental import pallas as pl
from jax.experimental.pallas import tpu as pltpu
from jax.experimental.pallas import tpu_sc as plsc

_D = 64
_NC = 2
_NS = 16
_NW = _NC * _NS
_CHUNK = 128


def _gather_body(idx_hbm, table_hbm, out_hbm, idx_v, rows_v, sem):
    wid = lax.axis_index("s") * _NC + lax.axis_index("c")
    nchunk = idx_hbm.shape[1]
    pltpu.sync_copy(idx_hbm.at[wid], idx_v)

    def body(c, carry):
        pltpu.async_copy(table_hbm.at[idx_v.at[c]], rows_v, sem).wait()
        pltpu.sync_copy(
            rows_v, out_hbm.at[pl.ds((wid * nchunk + c) * _CHUNK, _CHUNK)]
        )
        return carry

    lax.fori_loop(0, nchunk, body, 0)


def kernel(inputs, embeddings):
    b, h = inputs.shape
    total = b * h
    assert total % (_NW * _CHUNK) == 0
    nchunk = total // (_NW * _CHUNK)
    flat_idx = jnp.reshape(inputs.astype(jnp.int32), (_NW, nchunk, _CHUNK))
    mesh = plsc.VectorSubcoreMesh(core_axis_name="c", subcore_axis_name="s")
    fn = pl.kernel(
        _gather_body,
        mesh=mesh,
        out_type=jax.ShapeDtypeStruct((total, _D), jnp.float32),
        scratch_types=[
            pltpu.VMEM((nchunk, _CHUNK), jnp.int32),
            pltpu.VMEM((_CHUNK, _D), jnp.float32),
            pltpu.SemaphoreType.DMA,
        ],
        compiler_params=pltpu.CompilerParams(use_tc_tiling_on_sc=False),
    )
    out = fn(flat_idx, embeddings)
    return jnp.reshape(out, (b, h, _D))


# speedup vs baseline: 1.1150x; 1.1150x over previous
import jax
import jax.numpy as jnp
from jax import lax
from jax.experimental import pallas as pl
from jax.experimental.pallas import tpu as pltpu
from jax.experimental.pallas import tpu_sc as plsc

_D = 64
_NC = 2
_NS = 16
_NW = _NC * _NS
_CHUNK = 128
_K = 4
_GROUP_ROWS = _K * _CHUNK


def _gather_body(idx_hbm, table_hbm, out_hbm, idx_v, bufs, g0, g1, w0, w1):
    gsem = (g0, g1)
    wsem = (w0, w1)
    wid = lax.axis_index("s") * _NC + lax.axis_index("c")
    nchunk = idx_hbm.shape[1]
    ngrp = nchunk // _K
    base_row = wid * nchunk * _CHUNK
    pltpu.sync_copy(idx_hbm.at[wid], idx_v)

    def drain(sem, p):
        pltpu.make_async_copy(
            out_hbm.at[pl.ds(0, _GROUP_ROWS)], bufs.at[p], sem
        ).wait()

    def fire_gathers(g, p):
        for j in range(_K):
            pltpu.async_copy(
                table_hbm.at[idx_v.at[g * _K + j]],
                bufs.at[p].at[pl.ds(j * _CHUNK, _CHUNK)],
                gsem[p],
            )

    def fire_write(g, p):
        pltpu.async_copy(
            bufs.at[p],
            out_hbm.at[pl.ds(base_row + g * _GROUP_ROWS, _GROUP_ROWS)],
            wsem[p],
        )

    def outer(g2, carry):
        for p in range(2):
            g = g2 * 2 + p

            @pl.when(g >= 2)
            def _():
                drain(wsem[p], p)

            fire_gathers(g, p)

            @pl.when(g >= 1)
            def _():
                drain(gsem[1 - p], 1 - p)
                fire_write(g - 1, 1 - p)

        return carry

    lax.fori_loop(0, ngrp // 2, outer, 0)
    drain(gsem[1], 1)
    fire_write(ngrp - 1, 1)
    drain(wsem[0], 0)
    drain(wsem[1], 1)


def kernel(inputs, embeddings):
    b, h = inputs.shape
    total = b * h
    assert total % (_NW * _GROUP_ROWS) == 0
    nchunk = total // (_NW * _CHUNK)
    flat_idx = jnp.reshape(inputs.astype(jnp.int32), (_NW, nchunk, _CHUNK))
    mesh = plsc.VectorSubcoreMesh(core_axis_name="c", subcore_axis_name="s")
    fn = pl.kernel(
        _gather_body,
        mesh=mesh,
        out_type=jax.ShapeDtypeStruct((total, _D), jnp.float32),
        scratch_types=[
            pltpu.VMEM((nchunk, _CHUNK), jnp.int32),
            pltpu.VMEM((2, _GROUP_ROWS, _D), jnp.float32),
            pltpu.SemaphoreType.DMA,
            pltpu.SemaphoreType.DMA,
            pltpu.SemaphoreType.DMA,
            pltpu.SemaphoreType.DMA,
        ],
        compiler_params=pltpu.CompilerParams(use_tc_tiling_on_sc=False),
    )
    out = fn(flat_idx, embeddings)
    return jnp.reshape(out, (b, h, _D))
